# baseline (speedup 1.0000x reference)
import functools

import jax
import jax.numpy as jnp
from jax import lax
from jax.experimental import pallas as pl
from jax.experimental.pallas import tpu as pltpu
from jax.experimental.pallas import tpu_sc as plsc

N = 10000
E = 320000
D = 128

NC = 2
NS = 16
NW = NC * NS
LANES = 16

HALF = D // NC
CH = 128
NBUF = 2

N_PAD = 10240
ROWS_PT = N_PAD // NS

NCH = 160
EPT = NCH * CH
E_PAD = NS * EPT
EDT = E // NW

R_BLK = 1280

_mesh = plsc.VectorSubcoreMesh(
    core_axis_name="c", subcore_axis_name="s", num_cores=NC, num_subcores=NS
)


@functools.partial(
    pl.kernel,
    out_type=jax.ShapeDtypeStruct((NW, N_PAD), jnp.float32),
    mesh=_mesh,
    compiler_params=pltpu.CompilerParams(needs_layout_passes=False),
    scratch_types=[
        pltpu.VMEM((EDT,), jnp.int32),
        pltpu.VMEM((N_PAD,), jnp.float32),
    ],
)
def _deg_sc(dstd_hbm, out_hbm, idx_v, deg_v):
    c = lax.axis_index("c")
    s = lax.axis_index("s")
    w = s * NC + c
    pltpu.sync_copy(dstd_hbm.at[w], idx_v)

    zero16 = jnp.zeros((LANES,), jnp.float32)

    @pl.loop(0, N_PAD // LANES)
    def _(i):
        deg_v[pl.ds(i * LANES, LANES)] = zero16

    ones16 = jnp.ones((LANES,), jnp.float32)

    @pl.loop(0, EDT // LANES)
    def _(i):
        idx = idx_v[pl.ds(i * LANES, LANES)]
        plsc.addupdate_scatter(deg_v, [idx], ones16)

    pltpu.sync_copy(deg_v, out_hbm.at[w])


@functools.partial(
    pl.kernel,
    out_type=jax.ShapeDtypeStruct((NC, N_PAD, HALF), jnp.float32),
    mesh=_mesh,
    compiler_params=pltpu.CompilerParams(use_tc_tiling_on_sc=False),
    scratch_types=[
        pltpu.VMEM((NCH,), jnp.int32),
        pltpu.VMEM((NCH,), jnp.int32),
        pltpu.VMEM((NCH, CH), jnp.int32),
        pltpu.VMEM((NCH, CH), jnp.int32),
        pltpu.VMEM((CH, HALF), jnp.float32),
        pltpu.VMEM((NBUF, CH, HALF), jnp.float32),
        pltpu.VMEM_SHARED((N_PAD + 16, HALF), jnp.float32),
    ] + [pltpu.SemaphoreType.DMA] * (2 * NBUF + 1),
)
def _agg_sc(hwq_hbm, srcq_hbm, dstq_hbm, out_hbm,
            rid_src, rid_dst, src_v, dst_v, zbuf, bufs, agg_s, *sems):
    gsems = sems[:NBUF]
    ssems = sems[NBUF:2 * NBUF]
    isem = sems[2 * NBUF]
    c = lax.axis_index("c")
    s = lax.axis_index("s")
    base = s * ROWS_PT
    bsrc = (c * NS + s) * NCH
    bdst = s * NCH

    iota16 = lax.iota(jnp.int32, LANES)
    for k in range(NCH // LANES):
        rid_src[pl.ds(k * LANES, LANES)] = iota16 + (bsrc + k * LANES)
        rid_dst[pl.ds(k * LANES, LANES)] = iota16 + (bdst + k * LANES)

    pltpu.async_copy(srcq_hbm.at[rid_src.at[pl.ds(0, 128)]],
                     src_v.at[pl.ds(0, 128)], isem).wait()
    pltpu.async_copy(srcq_hbm.at[rid_src.at[pl.ds(128, NCH - 128)]],
                     src_v.at[pl.ds(128, NCH - 128)], isem).wait()
    pltpu.async_copy(dstq_hbm.at[rid_dst.at[pl.ds(0, 128)]],
                     dst_v.at[pl.ds(0, 128)], isem).wait()
    pltpu.async_copy(dstq_hbm.at[rid_dst.at[pl.ds(128, NCH - 128)]],
                     dst_v.at[pl.ds(128, NCH - 128)], isem).wait()

    zero16 = jnp.zeros((LANES,), jnp.float32)

    @pl.loop(0, CH)
    def _(r):
        for k in range(HALF // LANES):
            zbuf[r, pl.ds(k * LANES, LANES)] = zero16

    @pl.loop(0, ROWS_PT // CH)
    def _(t):
        pltpu.sync_copy(zbuf, agg_s.at[pl.ds(base + t * CH, CH)])

    @pl.when(s == 0)
    def _():
        pltpu.sync_copy(zbuf.at[pl.ds(0, 16)],
                        agg_s.at[pl.ds(N_PAD, 16)])

    plsc.subcore_barrier()

    for b in range(NBUF):
        pltpu.async_copy(hwq_hbm.at[src_v.at[b]], bufs.at[b], gsems[b])

    @pl.loop(0, NCH // NBUF)
    def _(g):
        for b in range(NBUF):
            j = g * NBUF + b
            pltpu.make_async_copy(hwq_hbm.at[src_v.at[j]], bufs.at[b],
                                  gsems[b]).wait()
            pltpu.async_copy(bufs.at[b], agg_s.at[dst_v.at[j]], ssems[b],
                             add=True)
            jn = j + NBUF

            @pl.when(jn < NCH)
            def _():
                pltpu.make_async_copy(bufs.at[b], agg_s.at[dst_v.at[j]],
                                      ssems[b]).wait()
                pltpu.async_copy(hwq_hbm.at[src_v.at[jn]], bufs.at[b],
                                 gsems[b])

    for b in range(NBUF):
        pltpu.make_async_copy(bufs.at[b], agg_s.at[dst_v.at[0]],
                              ssems[b]).wait()

    plsc.subcore_barrier()

    pltpu.sync_copy(agg_s.at[pl.ds(base, ROWS_PT)],
                    out_hbm.at[c].at[pl.ds(base, ROWS_PT)])


def _mm_body(h_ref, w_ref, o_ref):
    o_ref[...] = jnp.dot(h_ref[...], w_ref[0],
                         preferred_element_type=jnp.float32)


def _mm_tc(h, Wsplit):
    nb = N_PAD // R_BLK
    return pl.pallas_call(
        _mm_body,
        grid=(NC, nb),
        in_specs=[
            pl.BlockSpec((R_BLK, D), lambda c, i: (i, 0)),
            pl.BlockSpec((1, D, HALF), lambda c, i: (c, 0, 0)),
        ],
        out_specs=pl.BlockSpec((R_BLK, HALF),
                               lambda c, i: (c * (N_PAD // R_BLK) + i, 0)),
        out_shape=jax.ShapeDtypeStruct((NC * N_PAD, HALF), jnp.float32),
    )(h, Wsplit)


def _epi_body(a_ref, d_ref, b_ref, h_ref, o_ref, *, final):
    agg = jnp.concatenate([a_ref[0], a_ref[1]], axis=1)
    deg = jnp.sum(d_ref[...], axis=0)
    inv = 1.0 / jnp.maximum(deg, 1.0)
    out = h_ref[...] + jnp.maximum(agg * inv[:, None] + b_ref[0:1, :], 0.0)
    if final:
        nrm = jnp.sqrt(jnp.sum(out * out, axis=1, keepdims=True))
        out = out / jnp.maximum(nrm, 1e-12)
    o_ref[...] = out


def _epi_tc(aggp, degp, b8, h, final):
    nb = N_PAD // R_BLK
    return pl.pallas_call(
        functools.partial(_epi_body, final=final),
        grid=(nb,),
        in_specs=[
            pl.BlockSpec((NC, R_BLK, HALF), lambda i: (0, i, 0)),
            pl.BlockSpec((NW, R_BLK), lambda i: (0, i)),
            pl.BlockSpec((8, D), lambda i: (0, 0)),
            pl.BlockSpec((R_BLK, D), lambda i: (i, 0)),
        ],
        out_specs=pl.BlockSpec((R_BLK, D), lambda i: (i, 0)),
        out_shape=jax.ShapeDtypeStruct((N_PAD, D), jnp.float32),
    )(aggp, degp, b8, h)


def kernel(x, edge_index, W0, b0, W1, b1):
    src = edge_index[0]
    dst = edge_index[1]

    pad_e = E_PAD - E
    src_p = jnp.concatenate([src, jnp.zeros((pad_e,), jnp.int32)])
    dst_p = jnp.concatenate([dst, jnp.full((pad_e,), N_PAD, jnp.int32)])
    src3 = src_p.reshape(NS, NCH, CH)
    srcq = jnp.stack([src3, src3 + N_PAD]).reshape(NC * NS * NCH, CH)
    dstq = dst_p.reshape(NS * NCH, CH)
    dstd = dst.reshape(NW, EDT)

    h = jnp.pad(x, ((0, N_PAD - N), (0, 0)))
    b0_8 = jnp.broadcast_to(b0[None, :], (8, D))
    b1_8 = jnp.broadcast_to(b1[None, :], (8, D))
    W0s = jnp.stack([W0[:, :HALF], W0[:, HALF:]])
    W1s = jnp.stack([W1[:, :HALF], W1[:, HALF:]])

    degp = _deg_sc(dstd)

    hw0 = _mm_tc(h, W0s)
    agg0 = _agg_sc(hw0, srcq, dstq)
    h1 = _epi_tc(agg0, degp, b0_8, h, final=False)

    hw1 = _mm_tc(h1, W1s)
    agg1 = _agg_sc(hw1, srcq, dstq)
    out = _epi_tc(agg1, degp, b1_8, h1, final=True)

    return out[:N]


# speedup vs baseline: 1.6414x; 1.6414x over previous
import functools

import jax
import jax.numpy as jnp
from jax import lax
from jax.experimental import pallas as pl
from jax.experimental.pallas import tpu as pltpu
from jax.experimental.pallas import tpu_sc as plsc

N = 10000
E = 320000
D = 128

NC = 2
NS = 16
NW = NC * NS
LANES = 16

HALF = D // NC
CH = 128
NBUF = 4

N_PAD = 10240
ROWS_PT = N_PAD // NS

NCH = 160
EPT = NCH * CH
E_PAD = NS * EPT
EDT = E // NW

R_BLK = 1280

_mesh = plsc.VectorSubcoreMesh(
    core_axis_name="c", subcore_axis_name="s", num_cores=NC, num_subcores=NS
)


@functools.partial(
    pl.kernel,
    out_type=jax.ShapeDtypeStruct((NW, N_PAD), jnp.float32),
    mesh=_mesh,
    compiler_params=pltpu.CompilerParams(needs_layout_passes=False),
    scratch_types=[
        pltpu.VMEM((EDT,), jnp.int32),
        pltpu.VMEM((N_PAD,), jnp.float32),
    ],
)
def _deg_sc(dstd_hbm, out_hbm, idx_v, deg_v):
    c = lax.axis_index("c")
    s = lax.axis_index("s")
    w = s * NC + c
    pltpu.sync_copy(dstd_hbm.at[w], idx_v)

    zero16 = jnp.zeros((LANES,), jnp.float32)

    @pl.loop(0, N_PAD // LANES)
    def _(i):
        deg_v[pl.ds(i * LANES, LANES)] = zero16

    ones16 = jnp.ones((LANES,), jnp.float32)

    @pl.loop(0, EDT // LANES)
    def _(i):
        idx = idx_v[pl.ds(i * LANES, LANES)]
        plsc.addupdate_scatter(deg_v, [idx], ones16)

    pltpu.sync_copy(deg_v, out_hbm.at[w])


@functools.partial(
    pl.kernel,
    out_type=jax.ShapeDtypeStruct((NC, N_PAD, HALF), jnp.bfloat16),
    mesh=_mesh,
    compiler_params=pltpu.CompilerParams(use_tc_tiling_on_sc=False),
    scratch_types=[
        pltpu.VMEM((NCH,), jnp.int32),
        pltpu.VMEM((NCH,), jnp.int32),
        pltpu.VMEM((NCH, CH), jnp.int32),
        pltpu.VMEM((NCH, CH), jnp.int32),
        pltpu.VMEM((CH, HALF), jnp.bfloat16),
        pltpu.VMEM((NBUF, CH, HALF), jnp.bfloat16),
        pltpu.VMEM_SHARED((N_PAD + 16, HALF), jnp.bfloat16),
    ] + [pltpu.SemaphoreType.DMA] * (2 * NBUF + 1),
)
def _agg_sc(hwq_hbm, srcq_hbm, dstq_hbm, out_hbm,
            rid_src, rid_dst, src_v, dst_v, zbuf, bufs, agg_s, *sems):
    gsems = sems[:NBUF]
    ssems = sems[NBUF:2 * NBUF]
    isem = sems[2 * NBUF]
    c = lax.axis_index("c")
    s = lax.axis_index("s")
    base = s * ROWS_PT
    bsrc = (c * NS + s) * NCH
    bdst = s * NCH

    iota16 = lax.iota(jnp.int32, LANES)
    for k in range(NCH // LANES):
        rid_src[pl.ds(k * LANES, LANES)] = iota16 + (bsrc + k * LANES)
        rid_dst[pl.ds(k * LANES, LANES)] = iota16 + (bdst + k * LANES)

    pltpu.async_copy(srcq_hbm.at[rid_src.at[pl.ds(0, 128)]],
                     src_v.at[pl.ds(0, 128)], isem).wait()
    pltpu.async_copy(srcq_hbm.at[rid_src.at[pl.ds(128, NCH - 128)]],
                     src_v.at[pl.ds(128, NCH - 128)], isem).wait()
    pltpu.async_copy(dstq_hbm.at[rid_dst.at[pl.ds(0, 128)]],
                     dst_v.at[pl.ds(0, 128)], isem).wait()
    pltpu.async_copy(dstq_hbm.at[rid_dst.at[pl.ds(128, NCH - 128)]],
                     dst_v.at[pl.ds(128, NCH - 128)], isem).wait()

    zero32 = jnp.zeros((2 * LANES,), jnp.bfloat16)

    @pl.loop(0, CH)
    def _(r):
        for k in range(HALF // (2 * LANES)):
            zbuf[r, pl.ds(k * 2 * LANES, 2 * LANES)] = zero32

    @pl.loop(0, ROWS_PT // CH)
    def _(t):
        pltpu.sync_copy(zbuf, agg_s.at[pl.ds(base + t * CH, CH)])

    @pl.when(s == 0)
    def _():
        pltpu.sync_copy(zbuf.at[pl.ds(0, 16)],
                        agg_s.at[pl.ds(N_PAD, 16)])

    plsc.subcore_barrier()

    for b in range(NBUF):
        pltpu.async_copy(hwq_hbm.at[src_v.at[b]], bufs.at[b], gsems[b])

    @pl.loop(0, NCH // NBUF)
    def _(g):
        for b in range(NBUF):
            j = g * NBUF + b
            pltpu.make_async_copy(hwq_hbm.at[src_v.at[j]], bufs.at[b],
                                  gsems[b]).wait()
            pltpu.async_copy(bufs.at[b], agg_s.at[dst_v.at[j]], ssems[b],
                             add=True)
            jn = j + NBUF

            @pl.when(jn < NCH)
            def _():
                pltpu.make_async_copy(bufs.at[b], agg_s.at[dst_v.at[j]],
                                      ssems[b]).wait()
                pltpu.async_copy(hwq_hbm.at[src_v.at[jn]], bufs.at[b],
                                 gsems[b])

    for b in range(NBUF):
        pltpu.make_async_copy(bufs.at[b], agg_s.at[dst_v.at[0]],
                              ssems[b]).wait()

    plsc.subcore_barrier()

    pltpu.sync_copy(agg_s.at[pl.ds(base, ROWS_PT)],
                    out_hbm.at[c].at[pl.ds(base, ROWS_PT)])


def _mm_body(h_ref, w_ref, o_ref):
    o_ref[...] = jnp.dot(h_ref[...], w_ref[0],
                         preferred_element_type=jnp.float32).astype(jnp.bfloat16)


def _mm_tc(h, Wsplit):
    nb = N_PAD // R_BLK
    return pl.pallas_call(
        _mm_body,
        grid=(NC, nb),
        in_specs=[
            pl.BlockSpec((R_BLK, D), lambda c, i: (i, 0)),
            pl.BlockSpec((1, D, HALF), lambda c, i: (c, 0, 0)),
        ],
        out_specs=pl.BlockSpec((R_BLK, HALF),
                               lambda c, i: (c * (N_PAD // R_BLK) + i, 0)),
        out_shape=jax.ShapeDtypeStruct((NC * N_PAD, HALF), jnp.bfloat16),
    )(h, Wsplit)


def _epi_body(a_ref, d_ref, b_ref, h_ref, o_ref, *, final):
    agg = jnp.concatenate([a_ref[0], a_ref[1]], axis=1).astype(jnp.float32)
    deg = jnp.sum(d_ref[...], axis=0)
    inv = 1.0 / jnp.maximum(deg, 1.0)
    out = h_ref[...] + jnp.maximum(agg * inv[:, None] + b_ref[0:1, :], 0.0)
    if final:
        nrm = jnp.sqrt(jnp.sum(out * out, axis=1, keepdims=True))
        out = out / jnp.maximum(nrm, 1e-12)
    o_ref[...] = out


def _epi_tc(aggp, degp, b8, h, final):
    nb = N_PAD // R_BLK
    return pl.pallas_call(
        functools.partial(_epi_body, final=final),
        grid=(nb,),
        in_specs=[
            pl.BlockSpec((NC, R_BLK, HALF), lambda i: (0, i, 0)),
            pl.BlockSpec((NW, R_BLK), lambda i: (0, i)),
            pl.BlockSpec((8, D), lambda i: (0, 0)),
            pl.BlockSpec((R_BLK, D), lambda i: (i, 0)),
        ],
        out_specs=pl.BlockSpec((R_BLK, D), lambda i: (i, 0)),
        out_shape=jax.ShapeDtypeStruct((N_PAD, D), jnp.float32),
    )(aggp, degp, b8, h)


def kernel(x, edge_index, W0, b0, W1, b1):
    src = edge_index[0]
    dst = edge_index[1]

    pad_e = E_PAD - E
    src_p = jnp.concatenate([src, jnp.zeros((pad_e,), jnp.int32)])
    dst_p = jnp.concatenate([dst, jnp.full((pad_e,), N_PAD, jnp.int32)])
    src3 = src_p.reshape(NS, NCH, CH)
    srcq = jnp.stack([src3, src3 + N_PAD]).reshape(NC * NS * NCH, CH)
    dstq = dst_p.reshape(NS * NCH, CH)
    dstd = dst.reshape(NW, EDT)

    h = jnp.pad(x, ((0, N_PAD - N), (0, 0)))
    b0_8 = jnp.broadcast_to(b0[None, :], (8, D))
    b1_8 = jnp.broadcast_to(b1[None, :], (8, D))
    W0s = jnp.stack([W0[:, :HALF], W0[:, HALF:]])
    W1s = jnp.stack([W1[:, :HALF], W1[:, HALF:]])

    degp = _deg_sc(dstd)

    hw0 = _mm_tc(h, W0s)
    agg0 = _agg_sc(hw0, srcq, dstq)
    h1 = _epi_tc(agg0, degp, b0_8, h, final=False)

    hw1 = _mm_tc(h1, W1s)
    agg1 = _agg_sc(hw1, srcq, dstq)
    out = _epi_tc(agg1, degp, b1_8, h1, final=True)

    return out[:N]


# speedup vs baseline: 2.4922x; 1.5183x over previous
import functools

import jax
import jax.numpy as jnp
from jax import lax
from jax.experimental import pallas as pl
from jax.experimental.pallas import tpu as pltpu
from jax.experimental.pallas import tpu_sc as plsc

N = 10000
E = 320000
D = 128

NC = 2
NS = 16
NW = NC * NS
LANES = 16

HALF = D // NC
CH = 128
NBUF = 4

N_PAD = 10240
ROWS_PT = N_PAD // NS

NCH = 160
EPT = NCH * CH
E_PAD = NS * EPT
EDT = E // NW

R_BLK = 1280

_mesh = plsc.VectorSubcoreMesh(
    core_axis_name="c", subcore_axis_name="s", num_cores=NC, num_subcores=NS
)


@functools.partial(
    pl.kernel,
    out_type=jax.ShapeDtypeStruct((NW, N_PAD), jnp.float32),
    mesh=_mesh,
    compiler_params=pltpu.CompilerParams(needs_layout_passes=False),
    scratch_types=[
        pltpu.VMEM((EDT,), jnp.int32),
        pltpu.VMEM((N_PAD,), jnp.float32),
    ],
)
def _deg_sc(dstd_hbm, out_hbm, idx_v, deg_v):
    c = lax.axis_index("c")
    s = lax.axis_index("s")
    w = s * NC + c
    pltpu.sync_copy(dstd_hbm.at[w], idx_v)

    zero16 = jnp.zeros((LANES,), jnp.float32)

    @pl.loop(0, N_PAD // LANES)
    def _(i):
        deg_v[pl.ds(i * LANES, LANES)] = zero16

    ones16 = jnp.ones((LANES,), jnp.float32)

    @pl.loop(0, EDT // LANES)
    def _(i):
        idx = idx_v[pl.ds(i * LANES, LANES)]
        plsc.addupdate_scatter(deg_v, [idx], ones16)

    pltpu.sync_copy(deg_v, out_hbm.at[w])


@functools.partial(
    pl.kernel,
    out_type=jax.ShapeDtypeStruct((NC, N_PAD, HALF), jnp.bfloat16),
    mesh=_mesh,
    compiler_params=pltpu.CompilerParams(use_tc_tiling_on_sc=False),
    scratch_types=[
        pltpu.VMEM((NCH,), jnp.int32),
        pltpu.VMEM((NCH,), jnp.int32),
        pltpu.VMEM((NCH, CH), jnp.int32),
        pltpu.VMEM((NCH, CH), jnp.int32),
        pltpu.VMEM((CH, HALF), jnp.bfloat16),
        pltpu.VMEM((NBUF, CH, HALF), jnp.bfloat16),
        pltpu.VMEM_SHARED((N_PAD, HALF), jnp.bfloat16),
        pltpu.VMEM_SHARED((N_PAD + 16, HALF), jnp.bfloat16),
    ] + [pltpu.SemaphoreType.DMA] * (2 * NBUF + 1),
)
def _agg_sc(hwq_hbm, srcq_hbm, dstq_hbm, out_hbm,
            rid_src, rid_dst, src_v, dst_v, zbuf, bufs, tab_s, agg_s, *sems):
    gsems = sems[:NBUF]
    ssems = sems[NBUF:2 * NBUF]
    isem = sems[2 * NBUF]
    c = lax.axis_index("c")
    s = lax.axis_index("s")
    base = s * ROWS_PT
    bsrc = s * NCH
    bdst = s * NCH

    iota16 = lax.iota(jnp.int32, LANES)
    for k in range(NCH // LANES):
        rid_src[pl.ds(k * LANES, LANES)] = iota16 + (bsrc + k * LANES)
        rid_dst[pl.ds(k * LANES, LANES)] = iota16 + (bdst + k * LANES)

    pltpu.async_copy(srcq_hbm.at[rid_src.at[pl.ds(0, 128)]],
                     src_v.at[pl.ds(0, 128)], isem).wait()
    pltpu.async_copy(srcq_hbm.at[rid_src.at[pl.ds(128, NCH - 128)]],
                     src_v.at[pl.ds(128, NCH - 128)], isem).wait()
    pltpu.async_copy(dstq_hbm.at[rid_dst.at[pl.ds(0, 128)]],
                     dst_v.at[pl.ds(0, 128)], isem).wait()
    pltpu.async_copy(dstq_hbm.at[rid_dst.at[pl.ds(128, NCH - 128)]],
                     dst_v.at[pl.ds(128, NCH - 128)], isem).wait()

    @pl.loop(0, ROWS_PT // CH)
    def _(t):
        for k in range(CH // LANES):
            rid_dst[pl.ds(k * LANES, LANES)] = iota16 + (
                c * N_PAD + base + t * CH + k * LANES)
        pltpu.async_copy(hwq_hbm.at[rid_dst.at[pl.ds(0, 128)]],
                         bufs.at[0], isem).wait()
        pltpu.sync_copy(bufs.at[0], tab_s.at[pl.ds(base + t * CH, CH)])

    for k in range(NCH // LANES):
        rid_dst[pl.ds(k * LANES, LANES)] = iota16 + (bdst + k * LANES)

    zero32 = jnp.zeros((2 * LANES,), jnp.bfloat16)

    @pl.loop(0, CH)
    def _(r):
        for k in range(HALF // (2 * LANES)):
            zbuf[r, pl.ds(k * 2 * LANES, 2 * LANES)] = zero32

    @pl.loop(0, ROWS_PT // CH)
    def _(t):
        pltpu.sync_copy(zbuf, agg_s.at[pl.ds(base + t * CH, CH)])

    @pl.when(s == 0)
    def _():
        pltpu.sync_copy(zbuf.at[pl.ds(0, 16)],
                        agg_s.at[pl.ds(N_PAD, 16)])

    plsc.subcore_barrier()

    for b in range(NBUF):
        pltpu.async_copy(tab_s.at[src_v.at[b]], bufs.at[b], gsems[b])

    @pl.loop(0, NCH // NBUF)
    def _(g):
        for b in range(NBUF):
            j = g * NBUF + b
            pltpu.make_async_copy(tab_s.at[src_v.at[j]], bufs.at[b],
                                  gsems[b]).wait()
            pltpu.async_copy(bufs.at[b], agg_s.at[dst_v.at[j]], ssems[b],
                             add=True)
            jn = j + NBUF

            @pl.when(jn < NCH)
            def _():
                pltpu.make_async_copy(bufs.at[b], agg_s.at[dst_v.at[j]],
                                      ssems[b]).wait()
                pltpu.async_copy(tab_s.at[src_v.at[jn]], bufs.at[b],
                                 gsems[b])

    for b in range(NBUF):
        pltpu.make_async_copy(bufs.at[b], agg_s.at[dst_v.at[0]],
                              ssems[b]).wait()

    plsc.subcore_barrier()

    pltpu.sync_copy(agg_s.at[pl.ds(base, ROWS_PT)],
                    out_hbm.at[c].at[pl.ds(base, ROWS_PT)])


def _mm_body(h_ref, w_ref, o_ref):
    o_ref[...] = jnp.dot(h_ref[...], w_ref[0],
                         preferred_element_type=jnp.float32).astype(jnp.bfloat16)


def _mm_tc(h, Wsplit):
    nb = N_PAD // R_BLK
    return pl.pallas_call(
        _mm_body,
        grid=(NC, nb),
        in_specs=[
            pl.BlockSpec((R_BLK, D), lambda c, i: (i, 0)),
            pl.BlockSpec((1, D, HALF), lambda c, i: (c, 0, 0)),
        ],
        out_specs=pl.BlockSpec((R_BLK, HALF),
                               lambda c, i: (c * (N_PAD // R_BLK) + i, 0)),
        out_shape=jax.ShapeDtypeStruct((NC * N_PAD, HALF), jnp.bfloat16),
    )(h, Wsplit)


def _epi_body(a_ref, d_ref, b_ref, h_ref, o_ref, *, final):
    agg = jnp.concatenate([a_ref[0], a_ref[1]], axis=1).astype(jnp.float32)
    deg = jnp.sum(d_ref[...], axis=0)
    inv = 1.0 / jnp.maximum(deg, 1.0)
    out = h_ref[...] + jnp.maximum(agg * inv[:, None] + b_ref[0:1, :], 0.0)
    if final:
        nrm = jnp.sqrt(jnp.sum(out * out, axis=1, keepdims=True))
        out = out / jnp.maximum(nrm, 1e-12)
    o_ref[...] = out


def _epi_tc(aggp, degp, b8, h, final):
    nb = N_PAD // R_BLK
    return pl.pallas_call(
        functools.partial(_epi_body, final=final),
        grid=(nb,),
        in_specs=[
            pl.BlockSpec((NC, R_BLK, HALF), lambda i: (0, i, 0)),
            pl.BlockSpec((NW, R_BLK), lambda i: (0, i)),
            pl.BlockSpec((8, D), lambda i: (0, 0)),
            pl.BlockSpec((R_BLK, D), lambda i: (i, 0)),
        ],
        out_specs=pl.BlockSpec((R_BLK, D), lambda i: (i, 0)),
        out_shape=jax.ShapeDtypeStruct((N_PAD, D), jnp.float32),
    )(aggp, degp, b8, h)


def kernel(x, edge_index, W0, b0, W1, b1):
    src = edge_index[0]
    dst = edge_index[1]

    pad_e = E_PAD - E
    src_p = jnp.concatenate([src, jnp.zeros((pad_e,), jnp.int32)])
    dst_p = jnp.concatenate([dst, jnp.full((pad_e,), N_PAD, jnp.int32)])
    srcq = src_p.reshape(NS * NCH, CH)
    dstq = dst_p.reshape(NS * NCH, CH)
    dstd = dst.reshape(NW, EDT)

    h = jnp.pad(x, ((0, N_PAD - N), (0, 0)))
    b0_8 = jnp.broadcast_to(b0[None, :], (8, D))
    b1_8 = jnp.broadcast_to(b1[None, :], (8, D))
    W0s = jnp.stack([W0[:, :HALF], W0[:, HALF:]])
    W1s = jnp.stack([W1[:, :HALF], W1[:, HALF:]])

    degp = _deg_sc(dstd)

    hw0 = _mm_tc(h, W0s)
    agg0 = _agg_sc(hw0, srcq, dstq)
    h1 = _epi_tc(agg0, degp, b0_8, h, final=False)

    hw1 = _mm_tc(h1, W1s)
    agg1 = _agg_sc(hw1, srcq, dstq)
    out = _epi_tc(agg1, degp, b1_8, h1, final=True)

    return out[:N]


# speedup vs baseline: 2.6112x; 1.0477x over previous
import functools

import jax
import jax.numpy as jnp
from jax import lax
from jax.experimental import pallas as pl
from jax.experimental.pallas import tpu as pltpu
from jax.experimental.pallas import tpu_sc as plsc

N = 10000
E = 320000
D = 128

NC = 2
NS = 16
NW = NC * NS
LANES = 16
BF = jnp.bfloat16

HALF = D // NC
CH = 128
NBUF = 4

N_PAD = 10240
ROWS_PT = N_PAD // NS

NCH = 160
EPT = NCH * CH
E_PAD = NS * EPT

R_BLK = 1280

_mesh = plsc.VectorSubcoreMesh(
    core_axis_name="c", subcore_axis_name="s", num_cores=NC, num_subcores=NS
)


def _agg_impl(with_deg, hwq_hbm, srcq_hbm, dstq_hbm, out_hbm, degp_hbm,
              rid_src, rid_dst, src_v, dst_v, zbuf, bufs, deg_v,
              tab_s, agg_s, sems):
    gsems = sems[:NBUF]
    ssems = sems[NBUF:2 * NBUF]
    isem = sems[2 * NBUF]
    c = lax.axis_index("c")
    s = lax.axis_index("s")
    base = s * ROWS_PT
    bsrc = s * NCH
    bdst = s * NCH

    iota16 = lax.iota(jnp.int32, LANES)
    for k in range(NCH // LANES):
        rid_src[pl.ds(k * LANES, LANES)] = iota16 + (bsrc + k * LANES)
        rid_dst[pl.ds(k * LANES, LANES)] = iota16 + (bdst + k * LANES)

    pltpu.async_copy(srcq_hbm.at[rid_src.at[pl.ds(0, 128)]],
                     src_v.at[pl.ds(0, 128)], isem).wait()
    pltpu.async_copy(srcq_hbm.at[rid_src.at[pl.ds(128, NCH - 128)]],
                     src_v.at[pl.ds(128, NCH - 128)], isem).wait()
    pltpu.async_copy(dstq_hbm.at[rid_dst.at[pl.ds(0, 128)]],
                     dst_v.at[pl.ds(0, 128)], isem).wait()
    pltpu.async_copy(dstq_hbm.at[rid_dst.at[pl.ds(128, NCH - 128)]],
                     dst_v.at[pl.ds(128, NCH - 128)], isem).wait()

    @pl.loop(0, ROWS_PT // CH)
    def _(t):
        for k in range(CH // LANES):
            rid_dst[pl.ds(k * LANES, LANES)] = iota16 + (
                c * N_PAD + base + t * CH + k * LANES)
        pltpu.async_copy(hwq_hbm.at[rid_dst.at[pl.ds(0, 128)]],
                         bufs.at[0], isem).wait()
        pltpu.sync_copy(bufs.at[0], tab_s.at[pl.ds(base + t * CH, CH)])

    for k in range(NCH // LANES):
        rid_dst[pl.ds(k * LANES, LANES)] = iota16 + (bdst + k * LANES)

    zero32 = jnp.zeros((2 * LANES,), BF)

    @pl.loop(0, CH)
    def _(r):
        for k in range(HALF // (2 * LANES)):
            zbuf[r, pl.ds(k * 2 * LANES, 2 * LANES)] = zero32

    @pl.loop(0, ROWS_PT // CH)
    def _(t):
        pltpu.sync_copy(zbuf, agg_s.at[pl.ds(base + t * CH, CH)])

    @pl.when(s == 0)
    def _():
        pltpu.sync_copy(zbuf.at[pl.ds(0, 16)],
                        agg_s.at[pl.ds(N_PAD, 16)])

    if with_deg:
        zero16 = jnp.zeros((LANES,), jnp.float32)

        @pl.loop(0, (N_PAD + 16) // LANES)
        def _(i):
            deg_v[pl.ds(i * LANES, LANES)] = zero16

    plsc.subcore_barrier()

    ones16 = jnp.ones((LANES,), jnp.float32)

    for b in range(NBUF):
        pltpu.async_copy(tab_s.at[src_v.at[b]], bufs.at[b], gsems[b])

    @pl.loop(0, NCH // NBUF)
    def _(g):
        for b in range(NBUF):
            j = g * NBUF + b
            pltpu.make_async_copy(tab_s.at[src_v.at[j]], bufs.at[b],
                                  gsems[b]).wait()
            pltpu.async_copy(bufs.at[b], agg_s.at[dst_v.at[j]], ssems[b],
                             add=True)
            if with_deg:
                @pl.when(c == 0)
                def _():
                    for k in range(CH // LANES):
                        idx = dst_v[j, pl.ds(k * LANES, LANES)]
                        plsc.addupdate_scatter(deg_v, [idx], ones16)
            jn = j + NBUF

            @pl.when(jn < NCH)
            def _():
                pltpu.make_async_copy(bufs.at[b], agg_s.at[dst_v.at[j]],
                                      ssems[b]).wait()
                pltpu.async_copy(tab_s.at[src_v.at[jn]], bufs.at[b],
                                 gsems[b])

    for b in range(NBUF):
        pltpu.make_async_copy(bufs.at[b], agg_s.at[dst_v.at[0]],
                              ssems[b]).wait()

    plsc.subcore_barrier()

    pltpu.sync_copy(agg_s.at[pl.ds(base, ROWS_PT)],
                    out_hbm.at[c].at[pl.ds(base, ROWS_PT)])
    if with_deg:
        @pl.when(c == 0)
        def _():
            pltpu.sync_copy(deg_v.at[pl.ds(0, N_PAD)], degp_hbm.at[s])


def _make_agg(with_deg):
    outs = [jax.ShapeDtypeStruct((NC, N_PAD, HALF), BF)]
    if with_deg:
        outs.append(jax.ShapeDtypeStruct((NS, N_PAD), jnp.float32))

    @functools.partial(
        pl.kernel,
        out_type=tuple(outs),
        mesh=_mesh,
        compiler_params=pltpu.CompilerParams(use_tc_tiling_on_sc=False,
                                             needs_layout_passes=False),
        scratch_types=[
            pltpu.VMEM((NCH,), jnp.int32),
            pltpu.VMEM((NCH,), jnp.int32),
            pltpu.VMEM((NCH, CH), jnp.int32),
            pltpu.VMEM((NCH, CH), jnp.int32),
            pltpu.VMEM((CH, HALF), BF),
            pltpu.VMEM((NBUF, CH, HALF), BF),
            pltpu.VMEM((N_PAD + 16,), jnp.float32),
            pltpu.VMEM_SHARED((N_PAD, HALF), BF),
            pltpu.VMEM_SHARED((N_PAD + 16, HALF), BF),
        ] + [pltpu.SemaphoreType.DMA] * (2 * NBUF + 1),
    )
    def _k(hwq_hbm, srcq_hbm, dstq_hbm, *rest):
        if with_deg:
            out_hbm, degp_hbm = rest[0], rest[1]
            scr = rest[2:]
        else:
            out_hbm, degp_hbm = rest[0], None
            scr = rest[1:]
        (rid_src, rid_dst, src_v, dst_v, zbuf, bufs, deg_v,
         tab_s, agg_s) = scr[:9]
        sems = scr[9:]
        _agg_impl(with_deg, hwq_hbm, srcq_hbm, dstq_hbm, out_hbm, degp_hbm,
                  rid_src, rid_dst, src_v, dst_v, zbuf, bufs, deg_v,
                  tab_s, agg_s, sems)

    return _k


_agg_deg_sc = _make_agg(True)
_agg_sc = _make_agg(False)


def _cast_body(h_ref, o_ref):
    for c in range(NC):
        o_ref[c] = h_ref[:, c * HALF:(c + 1) * HALF].astype(BF)


def _cast_tc(h):
    nb = N_PAD // R_BLK
    return pl.pallas_call(
        _cast_body,
        grid=(nb,),
        in_specs=[pl.BlockSpec((R_BLK, D), lambda i: (i, 0))],
        out_specs=pl.BlockSpec((NC, R_BLK, HALF), lambda i: (0, i, 0)),
        out_shape=jax.ShapeDtypeStruct((NC, N_PAD, HALF), BF),
    )(h)


def _epi_body(a_ref, d_ref, b_ref, h_ref, w_ref, o_ref, p_ref, *, final):
    agg = jnp.concatenate([a_ref[0], a_ref[1]], axis=1).astype(jnp.float32)
    deg = jnp.sum(d_ref[...], axis=0)
    inv = 1.0 / jnp.maximum(deg, 1.0)
    mean = agg * inv[:, None]
    z = jnp.dot(mean, w_ref[...], preferred_element_type=jnp.float32)
    out = h_ref[...] + jnp.maximum(z + b_ref[0:1, :], 0.0)
    if final:
        nrm = jnp.sqrt(jnp.sum(out * out, axis=1, keepdims=True))
        out = out / jnp.maximum(nrm, 1e-12)
    o_ref[...] = out
    if p_ref is not None:
        for c in range(NC):
            p_ref[c] = out[:, c * HALF:(c + 1) * HALF].astype(BF)


def _epi_tc(aggp, degp, b8, h, W, final):
    nb = N_PAD // R_BLK
    in_specs = [
        pl.BlockSpec((NC, R_BLK, HALF), lambda i: (0, i, 0)),
        pl.BlockSpec((NS, R_BLK), lambda i: (0, i)),
        pl.BlockSpec((8, D), lambda i: (0, 0)),
        pl.BlockSpec((R_BLK, D), lambda i: (i, 0)),
        pl.BlockSpec((D, D), lambda i: (0, 0)),
    ]
    if final:
        return pl.pallas_call(
            functools.partial(
                lambda a, d, b, hh, w, o: _epi_body(
                    a, d, b, hh, w, o, None, final=True)),
            grid=(nb,),
            in_specs=in_specs,
            out_specs=pl.BlockSpec((R_BLK, D), lambda i: (i, 0)),
            out_shape=jax.ShapeDtypeStruct((N_PAD, D), jnp.float32),
        )(aggp, degp, b8, h, W)
    return pl.pallas_call(
        lambda a, d, b, hh, w, o, p: _epi_body(
            a, d, b, hh, w, o, p, final=False),
        grid=(nb,),
        in_specs=in_specs,
        out_specs=[
            pl.BlockSpec((R_BLK, D), lambda i: (i, 0)),
            pl.BlockSpec((NC, R_BLK, HALF), lambda i: (0, i, 0)),
        ],
        out_shape=[
            jax.ShapeDtypeStruct((N_PAD, D), jnp.float32),
            jax.ShapeDtypeStruct((NC, N_PAD, HALF), BF),
        ],
    )(aggp, degp, b8, h, W)


def kernel(x, edge_index, W0, b0, W1, b1):
    src = edge_index[0]
    dst = edge_index[1]

    pad_e = E_PAD - E
    src_p = jnp.concatenate([src, jnp.zeros((pad_e,), jnp.int32)])
    dst_p = jnp.concatenate([dst, jnp.full((pad_e,), N_PAD, jnp.int32)])
    srcq = src_p.reshape(NS * NCH, CH)
    dstq = dst_p.reshape(NS * NCH, CH)

    h = jnp.pad(x, ((0, N_PAD - N), (0, 0)))
    b0_8 = jnp.broadcast_to(b0[None, :], (8, D))
    b1_8 = jnp.broadcast_to(b1[None, :], (8, D))

    planes0 = _cast_tc(h).reshape(NC * N_PAD, HALF)
    agg0, degp = _agg_deg_sc(planes0, srcq, dstq)
    h1, planes1 = _epi_tc(agg0, degp, b0_8, h, W0, final=False)

    agg1, = _agg_sc(planes1.reshape(NC * N_PAD, HALF), srcq, dstq)
    out = _epi_tc(agg1, degp, b1_8, h1, W1, final=True)

    return out[:N]


# speedup vs baseline: 2.6141x; 1.0011x over previous
import functools

import jax
import jax.numpy as jnp
from jax import lax
from jax.experimental import pallas as pl
from jax.experimental.pallas import tpu as pltpu
from jax.experimental.pallas import tpu_sc as plsc

N = 10000
E = 320000
D = 128

NC = 2
NS = 16
NW = NC * NS
LANES = 16
BF = jnp.bfloat16

HALF = D // NC
CH = 128
NBUF = 8

N_PAD = 10240
ROWS_PT = N_PAD // NS

NCH = 160
EPT = NCH * CH
E_PAD = NS * EPT

R_BLK = 1280

_mesh = plsc.VectorSubcoreMesh(
    core_axis_name="c", subcore_axis_name="s", num_cores=NC, num_subcores=NS
)


def _agg_impl(with_deg, hwq_hbm, srcq_hbm, dstq_hbm, out_hbm, degp_hbm,
              rid_src, rid_dst, src_v, dst_v, zbuf, bufs, deg_v,
              tab_s, agg_s, sems):
    gsems = sems[:NBUF]
    ssems = sems[NBUF:2 * NBUF]
    isem = sems[2 * NBUF]
    c = lax.axis_index("c")
    s = lax.axis_index("s")
    base = s * ROWS_PT
    bsrc = s * NCH
    bdst = s * NCH

    iota16 = lax.iota(jnp.int32, LANES)
    for k in range(NCH // LANES):
        rid_src[pl.ds(k * LANES, LANES)] = iota16 + (bsrc + k * LANES)
        rid_dst[pl.ds(k * LANES, LANES)] = iota16 + (bdst + k * LANES)

    pltpu.async_copy(srcq_hbm.at[rid_src.at[pl.ds(0, 128)]],
                     src_v.at[pl.ds(0, 128)], isem).wait()
    pltpu.async_copy(srcq_hbm.at[rid_src.at[pl.ds(128, NCH - 128)]],
                     src_v.at[pl.ds(128, NCH - 128)], isem).wait()
    pltpu.async_copy(dstq_hbm.at[rid_dst.at[pl.ds(0, 128)]],
                     dst_v.at[pl.ds(0, 128)], isem).wait()
    pltpu.async_copy(dstq_hbm.at[rid_dst.at[pl.ds(128, NCH - 128)]],
                     dst_v.at[pl.ds(128, NCH - 128)], isem).wait()

    @pl.loop(0, ROWS_PT // CH)
    def _(t):
        for k in range(CH // LANES):
            rid_dst[pl.ds(k * LANES, LANES)] = iota16 + (
                c * N_PAD + base + t * CH + k * LANES)
        pltpu.async_copy(hwq_hbm.at[rid_dst.at[pl.ds(0, 128)]],
                         bufs.at[0], isem).wait()
        pltpu.sync_copy(bufs.at[0], tab_s.at[pl.ds(base + t * CH, CH)])

    for k in range(NCH // LANES):
        rid_dst[pl.ds(k * LANES, LANES)] = iota16 + (bdst + k * LANES)

    zero32 = jnp.zeros((2 * LANES,), BF)

    @pl.loop(0, CH)
    def _(r):
        for k in range(HALF // (2 * LANES)):
            zbuf[r, pl.ds(k * 2 * LANES, 2 * LANES)] = zero32

    @pl.loop(0, ROWS_PT // CH)
    def _(t):
        pltpu.sync_copy(zbuf, agg_s.at[pl.ds(base + t * CH, CH)])

    @pl.when(s == 0)
    def _():
        pltpu.sync_copy(zbuf.at[pl.ds(0, 16)],
                        agg_s.at[pl.ds(N_PAD, 16)])

    if with_deg:
        zero16 = jnp.zeros((LANES,), jnp.float32)

        @pl.loop(0, (N_PAD + 16) // LANES)
        def _(i):
            deg_v[pl.ds(i * LANES, LANES)] = zero16

    plsc.subcore_barrier()

    ones16 = jnp.ones((LANES,), jnp.float32)

    for b in range(NBUF):
        pltpu.async_copy(tab_s.at[src_v.at[b]], bufs.at[b], gsems[b])

    @pl.loop(0, NCH // NBUF)
    def _(g):
        for b in range(NBUF):
            j = g * NBUF + b
            pltpu.make_async_copy(tab_s.at[src_v.at[j]], bufs.at[b],
                                  gsems[b]).wait()
            pltpu.async_copy(bufs.at[b], agg_s.at[dst_v.at[j]], ssems[b],
                             add=True)
            if with_deg:
                @pl.when(c == 0)
                def _():
                    for k in range(CH // LANES):
                        idx = dst_v[j, pl.ds(k * LANES, LANES)]
                        plsc.addupdate_scatter(deg_v, [idx], ones16)
            jn = j + NBUF

            @pl.when(jn < NCH)
            def _():
                pltpu.make_async_copy(bufs.at[b], agg_s.at[dst_v.at[j]],
                                      ssems[b]).wait()
                pltpu.async_copy(tab_s.at[src_v.at[jn]], bufs.at[b],
                                 gsems[b])

    for b in range(NBUF):
        pltpu.make_async_copy(bufs.at[b], agg_s.at[dst_v.at[0]],
                              ssems[b]).wait()

    plsc.subcore_barrier()

    pltpu.sync_copy(agg_s.at[pl.ds(base, ROWS_PT)],
                    out_hbm.at[c].at[pl.ds(base, ROWS_PT)])
    if with_deg:
        @pl.when(c == 0)
        def _():
            pltpu.sync_copy(deg_v.at[pl.ds(0, N_PAD)], degp_hbm.at[s])


def _make_agg(with_deg):
    outs = [jax.ShapeDtypeStruct((NC, N_PAD, HALF), BF)]
    if with_deg:
        outs.append(jax.ShapeDtypeStruct((NS, N_PAD), jnp.float32))

    @functools.partial(
        pl.kernel,
        out_type=tuple(outs),
        mesh=_mesh,
        compiler_params=pltpu.CompilerParams(use_tc_tiling_on_sc=False,
                                             needs_layout_passes=False),
        scratch_types=[
            pltpu.VMEM((NCH,), jnp.int32),
            pltpu.VMEM((NCH,), jnp.int32),
            pltpu.VMEM((NCH, CH), jnp.int32),
            pltpu.VMEM((NCH, CH), jnp.int32),
            pltpu.VMEM((CH, HALF), BF),
            pltpu.VMEM((NBUF, CH, HALF), BF),
            pltpu.VMEM((N_PAD + 16,), jnp.float32),
            pltpu.VMEM_SHARED((N_PAD, HALF), BF),
            pltpu.VMEM_SHARED((N_PAD + 16, HALF), BF),
        ] + [pltpu.SemaphoreType.DMA] * (2 * NBUF + 1),
    )
    def _k(hwq_hbm, srcq_hbm, dstq_hbm, *rest):
        if with_deg:
            out_hbm, degp_hbm = rest[0], rest[1]
            scr = rest[2:]
        else:
            out_hbm, degp_hbm = rest[0], None
            scr = rest[1:]
        (rid_src, rid_dst, src_v, dst_v, zbuf, bufs, deg_v,
         tab_s, agg_s) = scr[:9]
        sems = scr[9:]
        _agg_impl(with_deg, hwq_hbm, srcq_hbm, dstq_hbm, out_hbm, degp_hbm,
                  rid_src, rid_dst, src_v, dst_v, zbuf, bufs, deg_v,
                  tab_s, agg_s, sems)

    return _k


_agg_deg_sc = _make_agg(True)
_agg_sc = _make_agg(False)


def _cast_body(h_ref, o_ref):
    for c in range(NC):
        o_ref[c] = h_ref[:, c * HALF:(c + 1) * HALF].astype(BF)


def _cast_tc(h):
    nb = N_PAD // R_BLK
    return pl.pallas_call(
        _cast_body,
        grid=(nb,),
        in_specs=[pl.BlockSpec((R_BLK, D), lambda i: (i, 0))],
        out_specs=pl.BlockSpec((NC, R_BLK, HALF), lambda i: (0, i, 0)),
        out_shape=jax.ShapeDtypeStruct((NC, N_PAD, HALF), BF),
    )(h)


def _epi_body(a_ref, d_ref, b_ref, h_ref, w_ref, o_ref, p_ref, *, final):
    agg = jnp.concatenate([a_ref[0], a_ref[1]], axis=1).astype(jnp.float32)
    deg = jnp.sum(d_ref[...], axis=0)
    inv = 1.0 / jnp.maximum(deg, 1.0)
    mean = agg * inv[:, None]
    z = jnp.dot(mean, w_ref[...], preferred_element_type=jnp.float32)
    out = h_ref[...] + jnp.maximum(z + b_ref[0:1, :], 0.0)
    if final:
        nrm = jnp.sqrt(jnp.sum(out * out, axis=1, keepdims=True))
        out = out / jnp.maximum(nrm, 1e-12)
    o_ref[...] = out
    if p_ref is not None:
        for c in range(NC):
            p_ref[c] = out[:, c * HALF:(c + 1) * HALF].astype(BF)


def _epi_tc(aggp, degp, b8, h, W, final):
    nb = N_PAD // R_BLK
    in_specs = [
        pl.BlockSpec((NC, R_BLK, HALF), lambda i: (0, i, 0)),
        pl.BlockSpec((NS, R_BLK), lambda i: (0, i)),
        pl.BlockSpec((8, D), lambda i: (0, 0)),
        pl.BlockSpec((R_BLK, D), lambda i: (i, 0)),
        pl.BlockSpec((D, D), lambda i: (0, 0)),
    ]
    if final:
        return pl.pallas_call(
            functools.partial(
                lambda a, d, b, hh, w, o: _epi_body(
                    a, d, b, hh, w, o, None, final=True)),
            grid=(nb,),
            in_specs=in_specs,
            out_specs=pl.BlockSpec((R_BLK, D), lambda i: (i, 0)),
            out_shape=jax.ShapeDtypeStruct((N_PAD, D), jnp.float32),
        )(aggp, degp, b8, h, W)
    return pl.pallas_call(
        lambda a, d, b, hh, w, o, p: _epi_body(
            a, d, b, hh, w, o, p, final=False),
        grid=(nb,),
        in_specs=in_specs,
        out_specs=[
            pl.BlockSpec((R_BLK, D), lambda i: (i, 0)),
            pl.BlockSpec((NC, R_BLK, HALF), lambda i: (0, i, 0)),
        ],
        out_shape=[
            jax.ShapeDtypeStruct((N_PAD, D), jnp.float32),
            jax.ShapeDtypeStruct((NC, N_PAD, HALF), BF),
        ],
    )(aggp, degp, b8, h, W)


def kernel(x, edge_index, W0, b0, W1, b1):
    src = edge_index[0]
    dst = edge_index[1]

    pad_e = E_PAD - E
    src_p = jnp.concatenate([src, jnp.zeros((pad_e,), jnp.int32)])
    dst_p = jnp.concatenate([dst, jnp.full((pad_e,), N_PAD, jnp.int32)])
    srcq = src_p.reshape(NS * NCH, CH)
    dstq = dst_p.reshape(NS * NCH, CH)

    h = jnp.pad(x, ((0, N_PAD - N), (0, 0)))
    b0_8 = jnp.broadcast_to(b0[None, :], (8, D))
    b1_8 = jnp.broadcast_to(b1[None, :], (8, D))

    planes0 = _cast_tc(h).reshape(NC * N_PAD, HALF)
    agg0, degp = _agg_deg_sc(planes0, srcq, dstq)
    h1, planes1 = _epi_tc(agg0, degp, b0_8, h, W0, final=False)

    agg1, = _agg_sc(planes1.reshape(NC * N_PAD, HALF), srcq, dstq)
    out = _epi_tc(agg1, degp, b1_8, h1, W1, final=True)

    return out[:N]


# speedup vs baseline: 2.7094x; 1.0365x over previous
import functools

import jax
import jax.numpy as jnp
from jax import lax
from jax.experimental import pallas as pl
from jax.experimental.pallas import tpu as pltpu
from jax.experimental.pallas import tpu_sc as plsc

N = 10000
E = 320000
D = 128

NC = 2
NS = 16
NW = NC * NS
LANES = 16
BF = jnp.bfloat16

HALF = D // NC
CH = 128
NBUF = 4

N_PAD = 10240
ROWS_PT = N_PAD // NS

NCH = 160
EPT = NCH * CH
E_PAD = NS * EPT

R_BLK = 1280

_mesh = plsc.VectorSubcoreMesh(
    core_axis_name="c", subcore_axis_name="s", num_cores=NC, num_subcores=NS
)


def _agg_impl(with_deg, hwq_hbm, srcq_hbm, dstq_hbm, out_hbm, degp_hbm,
              rid_src, rid_dst, src_v, dst_v, rid_tab, zbuf, bufs, deg_v,
              tab_s, agg_s, sems):
    gsems = sems[:NBUF]
    ssems = sems[NBUF:2 * NBUF]
    isem = sems[2 * NBUF]
    c = lax.axis_index("c")
    s = lax.axis_index("s")
    base = s * ROWS_PT
    bsrc = s * NCH
    bdst = s * NCH

    iota16 = lax.iota(jnp.int32, LANES)
    for k in range(NCH // LANES):
        rid_src[pl.ds(k * LANES, LANES)] = iota16 + (bsrc + k * LANES)
        rid_dst[pl.ds(k * LANES, LANES)] = iota16 + (bdst + k * LANES)

    pltpu.async_copy(srcq_hbm.at[rid_src.at[pl.ds(0, 128)]],
                     src_v.at[pl.ds(0, 128)], isem)
    pltpu.async_copy(srcq_hbm.at[rid_src.at[pl.ds(128, NCH - 128)]],
                     src_v.at[pl.ds(128, NCH - 128)], isem)
    pltpu.async_copy(dstq_hbm.at[rid_dst.at[pl.ds(0, 128)]],
                     dst_v.at[pl.ds(0, 128)], isem)
    pltpu.async_copy(dstq_hbm.at[rid_dst.at[pl.ds(128, NCH - 128)]],
                     dst_v.at[pl.ds(128, NCH - 128)], isem)

    NST = ROWS_PT // CH
    for t in range(NST):
        for k in range(CH // LANES):
            rid_tab[t, pl.ds(k * LANES, LANES)] = iota16 + (
                c * N_PAD + base + t * CH + k * LANES)
    for t in range(2):
        pltpu.async_copy(hwq_hbm.at[rid_tab.at[t]], bufs.at[t], gsems[t])
    for t in range(NST):
        pltpu.make_async_copy(hwq_hbm.at[rid_tab.at[t]], bufs.at[t % 2],
                              gsems[t % 2]).wait()
        pltpu.sync_copy(bufs.at[t % 2], tab_s.at[pl.ds(base + t * CH, CH)])
        if t + 2 < NST:
            pltpu.async_copy(hwq_hbm.at[rid_tab.at[t + 2]],
                             bufs.at[t % 2], gsems[t % 2])

    zero32 = jnp.zeros((2 * LANES,), BF)

    @pl.loop(0, CH)
    def _(r):
        for k in range(HALF // (2 * LANES)):
            zbuf[r, pl.ds(k * 2 * LANES, 2 * LANES)] = zero32

    @pl.loop(0, ROWS_PT // CH)
    def _(t):
        pltpu.sync_copy(zbuf, agg_s.at[pl.ds(base + t * CH, CH)])

    @pl.when(s == 0)
    def _():
        pltpu.sync_copy(zbuf.at[pl.ds(0, 16)],
                        agg_s.at[pl.ds(N_PAD, 16)])

    if with_deg:
        zero16 = jnp.zeros((LANES,), jnp.float32)

        @pl.loop(0, (N_PAD + 16) // LANES)
        def _(i):
            deg_v[pl.ds(i * LANES, LANES)] = zero16

    pltpu.make_async_copy(srcq_hbm.at[rid_src.at[pl.ds(0, 128)]],
                          src_v.at[pl.ds(0, 128)], isem).wait()
    pltpu.make_async_copy(srcq_hbm.at[rid_src.at[pl.ds(128, NCH - 128)]],
                          src_v.at[pl.ds(128, NCH - 128)], isem).wait()
    pltpu.make_async_copy(dstq_hbm.at[rid_dst.at[pl.ds(0, 128)]],
                          dst_v.at[pl.ds(0, 128)], isem).wait()
    pltpu.make_async_copy(dstq_hbm.at[rid_dst.at[pl.ds(128, NCH - 128)]],
                          dst_v.at[pl.ds(128, NCH - 128)], isem).wait()

    plsc.subcore_barrier()

    ones16 = jnp.ones((LANES,), jnp.float32)

    for b in range(NBUF):
        pltpu.async_copy(tab_s.at[src_v.at[b]], bufs.at[b], gsems[b])

    @pl.loop(0, NCH // NBUF)
    def _(g):
        for b in range(NBUF):
            j = g * NBUF + b
            pltpu.make_async_copy(tab_s.at[src_v.at[j]], bufs.at[b],
                                  gsems[b]).wait()
            pltpu.async_copy(bufs.at[b], agg_s.at[dst_v.at[j]], ssems[b],
                             add=True)
            if with_deg:
                @pl.when(c == 0)
                def _():
                    for k in range(CH // LANES):
                        idx = dst_v[j, pl.ds(k * LANES, LANES)]
                        plsc.addupdate_scatter(deg_v, [idx], ones16)
            jn = j + NBUF

            @pl.when(jn < NCH)
            def _():
                pltpu.make_async_copy(bufs.at[b], agg_s.at[dst_v.at[j]],
                                      ssems[b]).wait()
                pltpu.async_copy(tab_s.at[src_v.at[jn]], bufs.at[b],
                                 gsems[b])

    for b in range(NBUF):
        pltpu.make_async_copy(bufs.at[b], agg_s.at[dst_v.at[0]],
                              ssems[b]).wait()

    plsc.subcore_barrier()

    pltpu.sync_copy(agg_s.at[pl.ds(base, ROWS_PT)],
                    out_hbm.at[c].at[pl.ds(base, ROWS_PT)])
    if with_deg:
        @pl.when(c == 0)
        def _():
            pltpu.sync_copy(deg_v.at[pl.ds(0, N_PAD)], degp_hbm.at[s])


def _make_agg(with_deg):
    outs = [jax.ShapeDtypeStruct((NC, N_PAD, HALF), BF)]
    if with_deg:
        outs.append(jax.ShapeDtypeStruct((NS, N_PAD), jnp.float32))

    @functools.partial(
        pl.kernel,
        out_type=tuple(outs),
        mesh=_mesh,
        compiler_params=pltpu.CompilerParams(use_tc_tiling_on_sc=False,
                                             needs_layout_passes=False),
        scratch_types=[
            pltpu.VMEM((NCH,), jnp.int32),
            pltpu.VMEM((NCH,), jnp.int32),
            pltpu.VMEM((NCH, CH), jnp.int32),
            pltpu.VMEM((NCH, CH), jnp.int32),
            pltpu.VMEM((ROWS_PT // CH, CH), jnp.int32),
            pltpu.VMEM((CH, HALF), BF),
            pltpu.VMEM((NBUF, CH, HALF), BF),
            pltpu.VMEM((N_PAD + 16,), jnp.float32),
            pltpu.VMEM_SHARED((N_PAD, HALF), BF),
            pltpu.VMEM_SHARED((N_PAD + 16, HALF), BF),
        ] + [pltpu.SemaphoreType.DMA] * (2 * NBUF + 1),
    )
    def _k(hwq_hbm, srcq_hbm, dstq_hbm, *rest):
        if with_deg:
            out_hbm, degp_hbm = rest[0], rest[1]
            scr = rest[2:]
        else:
            out_hbm, degp_hbm = rest[0], None
            scr = rest[1:]
        (rid_src, rid_dst, src_v, dst_v, rid_tab, zbuf, bufs, deg_v,
         tab_s, agg_s) = scr[:10]
        sems = scr[10:]
        _agg_impl(with_deg, hwq_hbm, srcq_hbm, dstq_hbm, out_hbm, degp_hbm,
                  rid_src, rid_dst, src_v, dst_v, rid_tab, zbuf, bufs, deg_v,
                  tab_s, agg_s, sems)

    return _k


_agg_deg_sc = _make_agg(True)
_agg_sc = _make_agg(False)


def _cast_body(h_ref, o_ref):
    for c in range(NC):
        o_ref[c] = h_ref[:, c * HALF:(c + 1) * HALF].astype(BF)


def _cast_tc(h):
    nb = N_PAD // R_BLK
    return pl.pallas_call(
        _cast_body,
        grid=(nb,),
        in_specs=[pl.BlockSpec((R_BLK, D), lambda i: (i, 0))],
        out_specs=pl.BlockSpec((NC, R_BLK, HALF), lambda i: (0, i, 0)),
        out_shape=jax.ShapeDtypeStruct((NC, N_PAD, HALF), BF),
    )(h)


def _epi_body(a_ref, d_ref, b_ref, h_ref, w_ref, o_ref, p_ref, *, final):
    agg = jnp.concatenate([a_ref[0], a_ref[1]], axis=1).astype(jnp.float32)
    deg = jnp.sum(d_ref[...], axis=0)
    inv = 1.0 / jnp.maximum(deg, 1.0)
    mean = agg * inv[:, None]
    z = jnp.dot(mean, w_ref[...], preferred_element_type=jnp.float32)
    out = h_ref[...] + jnp.maximum(z + b_ref[0:1, :], 0.0)
    if final:
        nrm = jnp.sqrt(jnp.sum(out * out, axis=1, keepdims=True))
        out = out / jnp.maximum(nrm, 1e-12)
    o_ref[...] = out
    if p_ref is not None:
        for c in range(NC):
            p_ref[c] = out[:, c * HALF:(c + 1) * HALF].astype(BF)


def _epi_tc(aggp, degp, b8, h, W, final):
    nb = N_PAD // R_BLK
    in_specs = [
        pl.BlockSpec((NC, R_BLK, HALF), lambda i: (0, i, 0)),
        pl.BlockSpec((NS, R_BLK), lambda i: (0, i)),
        pl.BlockSpec((8, D), lambda i: (0, 0)),
        pl.BlockSpec((R_BLK, D), lambda i: (i, 0)),
        pl.BlockSpec((D, D), lambda i: (0, 0)),
    ]
    if final:
        return pl.pallas_call(
            functools.partial(
                lambda a, d, b, hh, w, o: _epi_body(
                    a, d, b, hh, w, o, None, final=True)),
            grid=(nb,),
            in_specs=in_specs,
            out_specs=pl.BlockSpec((R_BLK, D), lambda i: (i, 0)),
            out_shape=jax.ShapeDtypeStruct((N_PAD, D), jnp.float32),
        )(aggp, degp, b8, h, W)
    return pl.pallas_call(
        lambda a, d, b, hh, w, o, p: _epi_body(
            a, d, b, hh, w, o, p, final=False),
        grid=(nb,),
        in_specs=in_specs,
        out_specs=[
            pl.BlockSpec((R_BLK, D), lambda i: (i, 0)),
            pl.BlockSpec((NC, R_BLK, HALF), lambda i: (0, i, 0)),
        ],
        out_shape=[
            jax.ShapeDtypeStruct((N_PAD, D), jnp.float32),
            jax.ShapeDtypeStruct((NC, N_PAD, HALF), BF),
        ],
    )(aggp, degp, b8, h, W)


def kernel(x, edge_index, W0, b0, W1, b1):
    src = edge_index[0]
    dst = edge_index[1]

    pad_e = E_PAD - E
    src_p = jnp.concatenate([src, jnp.zeros((pad_e,), jnp.int32)])
    dst_p = jnp.concatenate([dst, jnp.full((pad_e,), N_PAD, jnp.int32)])
    srcq = src_p.reshape(NS * NCH, CH)
    dstq = dst_p.reshape(NS * NCH, CH)

    h = jnp.pad(x, ((0, N_PAD - N), (0, 0)))
    b0_8 = jnp.broadcast_to(b0[None, :], (8, D))
    b1_8 = jnp.broadcast_to(b1[None, :], (8, D))

    planes0 = _cast_tc(h).reshape(NC * N_PAD, HALF)
    agg0, degp = _agg_deg_sc(planes0, srcq, dstq)
    h1, planes1 = _epi_tc(agg0, degp, b0_8, h, W0, final=False)

    agg1, = _agg_sc(planes1.reshape(NC * N_PAD, HALF), srcq, dstq)
    out = _epi_tc(agg1, degp, b1_8, h1, W1, final=True)

    return out[:N]


# speedup vs baseline: 2.7534x; 1.0162x over previous
import functools

import jax
import jax.numpy as jnp
from jax import lax
from jax.experimental import pallas as pl
from jax.experimental.pallas import tpu as pltpu
from jax.experimental.pallas import tpu_sc as plsc

N = 10000
E = 320000
D = 128

NC = 2
NS = 16
NW = NC * NS
LANES = 16
BF = jnp.bfloat16

HALF = D // NC
CH = 128
NBUF = 4

N_PAD = 10240
ROWS_PT = N_PAD // NS

NCH = 160
EPT = NCH * CH
E_PAD = NS * EPT

R_BLK = 1280

_mesh = plsc.VectorSubcoreMesh(
    core_axis_name="c", subcore_axis_name="s", num_cores=NC, num_subcores=NS
)


def _agg_impl(with_deg, hwq_hbm, srcq_hbm, dstq_hbm, out_hbm, degp_hbm,
              rid_src, rid_dst, src_v, dst_v, rid_tab, zbuf, bufs, deg_v,
              tab_s, agg_s, sems):
    gsems = sems[:NBUF]
    ssems = sems[NBUF:2 * NBUF]
    isem = sems[2 * NBUF]
    c = lax.axis_index("c")
    s = lax.axis_index("s")
    base = s * ROWS_PT
    bsrc = s * NCH
    bdst = s * NCH

    iota16 = lax.iota(jnp.int32, LANES)
    for k in range(NCH // LANES):
        rid_src[pl.ds(k * LANES, LANES)] = iota16 + (bsrc + k * LANES)
        rid_dst[pl.ds(k * LANES, LANES)] = iota16 + (bdst + k * LANES)

    pltpu.async_copy(srcq_hbm.at[rid_src.at[pl.ds(0, 128)]],
                     src_v.at[pl.ds(0, 128)], isem)
    pltpu.async_copy(srcq_hbm.at[rid_src.at[pl.ds(128, NCH - 128)]],
                     src_v.at[pl.ds(128, NCH - 128)], isem)
    pltpu.async_copy(dstq_hbm.at[rid_dst.at[pl.ds(0, 128)]],
                     dst_v.at[pl.ds(0, 128)], isem)
    pltpu.async_copy(dstq_hbm.at[rid_dst.at[pl.ds(128, NCH - 128)]],
                     dst_v.at[pl.ds(128, NCH - 128)], isem)

    NST = ROWS_PT // CH
    for t in range(NST):
        for k in range(CH // LANES):
            rid_tab[t, pl.ds(k * LANES, LANES)] = iota16 + (
                c * N_PAD + base + t * CH + k * LANES)
    for t in range(2):
        pltpu.async_copy(hwq_hbm.at[rid_tab.at[t]], bufs.at[t], gsems[t])
    for t in range(NST):
        pltpu.make_async_copy(hwq_hbm.at[rid_tab.at[t]], bufs.at[t % 2],
                              gsems[t % 2]).wait()
        pltpu.sync_copy(bufs.at[t % 2], tab_s.at[pl.ds(base + t * CH, CH)])
        if t + 2 < NST:
            pltpu.async_copy(hwq_hbm.at[rid_tab.at[t + 2]],
                             bufs.at[t % 2], gsems[t % 2])

    zero32 = jnp.zeros((2 * LANES,), BF)

    @pl.loop(0, CH)
    def _(r):
        for k in range(HALF // (2 * LANES)):
            zbuf[r, pl.ds(k * 2 * LANES, 2 * LANES)] = zero32

    @pl.loop(0, ROWS_PT // CH)
    def _(t):
        pltpu.sync_copy(zbuf, agg_s.at[pl.ds(base + t * CH, CH)])

    @pl.when(s == 0)
    def _():
        pltpu.sync_copy(zbuf.at[pl.ds(0, 16)],
                        agg_s.at[pl.ds(N_PAD, 16)])

    if with_deg:
        zero16 = jnp.zeros((LANES,), jnp.float32)

        @pl.loop(0, (N_PAD + 16) // LANES)
        def _(i):
            deg_v[pl.ds(i * LANES, LANES)] = zero16

    pltpu.make_async_copy(srcq_hbm.at[rid_src.at[pl.ds(0, 128)]],
                          src_v.at[pl.ds(0, 128)], isem).wait()
    pltpu.make_async_copy(srcq_hbm.at[rid_src.at[pl.ds(128, NCH - 128)]],
                          src_v.at[pl.ds(128, NCH - 128)], isem).wait()
    pltpu.make_async_copy(dstq_hbm.at[rid_dst.at[pl.ds(0, 128)]],
                          dst_v.at[pl.ds(0, 128)], isem).wait()
    pltpu.make_async_copy(dstq_hbm.at[rid_dst.at[pl.ds(128, NCH - 128)]],
                          dst_v.at[pl.ds(128, NCH - 128)], isem).wait()

    plsc.subcore_barrier()

    ones16 = jnp.ones((LANES,), jnp.float32)

    for b in range(NBUF):
        pltpu.async_copy(tab_s.at[src_v.at[b]], bufs.at[b], gsems[b])

    @pl.loop(0, NCH // NBUF)
    def _(g):
        for b in range(NBUF):
            j = g * NBUF + b
            pltpu.make_async_copy(tab_s.at[src_v.at[j]], bufs.at[b],
                                  gsems[b]).wait()
            pltpu.async_copy(bufs.at[b], agg_s.at[dst_v.at[j]], ssems[b],
                             add=True)
            if with_deg:
                @pl.when(c == 0)
                def _():
                    for k in range(CH // LANES):
                        idx = dst_v[j, pl.ds(k * LANES, LANES)]
                        plsc.addupdate_scatter(deg_v, [idx], ones16)
            jn = j + NBUF

            @pl.when(jn < NCH)
            def _():
                pltpu.make_async_copy(bufs.at[b], agg_s.at[dst_v.at[j]],
                                      ssems[b]).wait()
                pltpu.async_copy(tab_s.at[src_v.at[jn]], bufs.at[b],
                                 gsems[b])

    for b in range(NBUF):
        pltpu.make_async_copy(bufs.at[b], agg_s.at[dst_v.at[0]],
                              ssems[b]).wait()

    plsc.subcore_barrier()

    pltpu.sync_copy(agg_s.at[pl.ds(base, ROWS_PT)],
                    out_hbm.at[c].at[pl.ds(base, ROWS_PT)])
    if with_deg:
        @pl.when(c == 0)
        def _():
            pltpu.sync_copy(deg_v.at[pl.ds(0, N_PAD)], degp_hbm.at[s])


def _make_agg(with_deg):
    outs = [jax.ShapeDtypeStruct((NC, N_PAD, HALF), BF)]
    if with_deg:
        outs.append(jax.ShapeDtypeStruct((NS, N_PAD), jnp.float32))

    @functools.partial(
        pl.kernel,
        out_type=tuple(outs),
        mesh=_mesh,
        compiler_params=pltpu.CompilerParams(use_tc_tiling_on_sc=False,
                                             needs_layout_passes=False),
        scratch_types=[
            pltpu.VMEM((NCH,), jnp.int32),
            pltpu.VMEM((NCH,), jnp.int32),
            pltpu.VMEM((NCH, CH), jnp.int32),
            pltpu.VMEM((NCH, CH), jnp.int32),
            pltpu.VMEM((ROWS_PT // CH, CH), jnp.int32),
            pltpu.VMEM((CH, HALF), BF),
            pltpu.VMEM((NBUF, CH, HALF), BF),
            pltpu.VMEM((N_PAD + 16,), jnp.float32),
            pltpu.VMEM_SHARED((N_PAD, HALF), BF),
            pltpu.VMEM_SHARED((N_PAD + 16, HALF), BF),
        ] + [pltpu.SemaphoreType.DMA] * (2 * NBUF + 1),
    )
    def _k(hwq_hbm, srcq_hbm, dstq_hbm, *rest):
        if with_deg:
            out_hbm, degp_hbm = rest[0], rest[1]
            scr = rest[2:]
        else:
            out_hbm, degp_hbm = rest[0], None
            scr = rest[1:]
        (rid_src, rid_dst, src_v, dst_v, rid_tab, zbuf, bufs, deg_v,
         tab_s, agg_s) = scr[:10]
        sems = scr[10:]
        _agg_impl(with_deg, hwq_hbm, srcq_hbm, dstq_hbm, out_hbm, degp_hbm,
                  rid_src, rid_dst, src_v, dst_v, rid_tab, zbuf, bufs, deg_v,
                  tab_s, agg_s, sems)

    return _k


_agg_deg_sc = _make_agg(True)
_agg_sc = _make_agg(False)


def _cast_body(h_ref, o_ref):
    for c in range(NC):
        o_ref[c] = h_ref[:, c * HALF:(c + 1) * HALF].astype(BF)


def _cast_tc(h):
    nb = N_PAD // R_BLK
    return pl.pallas_call(
        _cast_body,
        grid=(nb,),
        in_specs=[pl.BlockSpec((R_BLK, D), lambda i: (i, 0))],
        out_specs=pl.BlockSpec((NC, R_BLK, HALF), lambda i: (0, i, 0)),
        out_shape=jax.ShapeDtypeStruct((NC, N_PAD, HALF), BF),
    )(h)


def _epi_body(a_ref, d_ref, b_ref, h_ref, w_ref, o_ref, p_ref, *, final):
    agg = jnp.concatenate([a_ref[0], a_ref[1]], axis=1).astype(jnp.float32)
    deg = jnp.sum(d_ref[...], axis=0)
    inv = 1.0 / jnp.maximum(deg, 1.0)
    mean = agg * inv[:, None]
    z = jnp.dot(mean, w_ref[...], preferred_element_type=jnp.float32)
    out = h_ref[...] + jnp.maximum(z + b_ref[0:1, :], 0.0)
    if final:
        nrm = jnp.sqrt(jnp.sum(out * out, axis=1, keepdims=True))
        out = out / jnp.maximum(nrm, 1e-12)
    o_ref[...] = out
    if p_ref is not None:
        for c in range(NC):
            p_ref[c] = out[:, c * HALF:(c + 1) * HALF].astype(BF)


def _epi_tc(aggp, degp, b8, h, W, final):
    nb = N_PAD // R_BLK
    in_specs = [
        pl.BlockSpec((NC, R_BLK, HALF), lambda i: (0, i, 0)),
        pl.BlockSpec((NS, R_BLK), lambda i: (0, i)),
        pl.BlockSpec((8, D), lambda i: (0, 0)),
        pl.BlockSpec((R_BLK, D), lambda i: (i, 0)),
        pl.BlockSpec((D, D), lambda i: (0, 0)),
    ]
    if final:
        return pl.pallas_call(
            functools.partial(
                lambda a, d, b, hh, w, o: _epi_body(
                    a, d, b, hh, w, o, None, final=True)),
            grid=(nb,),
            in_specs=in_specs,
            out_specs=pl.BlockSpec((R_BLK, D), lambda i: (i, 0)),
            out_shape=jax.ShapeDtypeStruct((N, D), jnp.float32),
        )(aggp, degp, b8, h, W)
    return pl.pallas_call(
        lambda a, d, b, hh, w, o, p: _epi_body(
            a, d, b, hh, w, o, p, final=False),
        grid=(nb,),
        in_specs=in_specs,
        out_specs=[
            pl.BlockSpec((R_BLK, D), lambda i: (i, 0)),
            pl.BlockSpec((NC, R_BLK, HALF), lambda i: (0, i, 0)),
        ],
        out_shape=[
            jax.ShapeDtypeStruct((N_PAD, D), jnp.float32),
            jax.ShapeDtypeStruct((NC, N_PAD, HALF), BF),
        ],
    )(aggp, degp, b8, h, W)


def kernel(x, edge_index, W0, b0, W1, b1):
    src = edge_index[0]
    dst = edge_index[1]

    pad_e = E_PAD - E
    src_p = jnp.concatenate([src, jnp.zeros((pad_e,), jnp.int32)])
    dst_p = jnp.concatenate([dst, jnp.full((pad_e,), N_PAD, jnp.int32)])
    srcq = src_p.reshape(NS * NCH, CH)
    dstq = dst_p.reshape(NS * NCH, CH)

    h = jnp.pad(x, ((0, N_PAD - N), (0, 0)))
    b0_8 = jnp.broadcast_to(b0[None, :], (8, D))
    b1_8 = jnp.broadcast_to(b1[None, :], (8, D))

    planes0 = _cast_tc(h).reshape(NC * N_PAD, HALF)
    agg0, degp = _agg_deg_sc(planes0, srcq, dstq)
    h1, planes1 = _epi_tc(agg0, degp, b0_8, h, W0, final=False)

    agg1, = _agg_sc(planes1.reshape(NC * N_PAD, HALF), srcq, dstq)
    out = _epi_tc(agg1, degp, b1_8, h1, W1, final=True)

    return out
